# baseline (device time: 153702 ns/iter reference)
import jax
import jax.numpy as jnp
from jax import lax
from jax.experimental import pallas as pl
from jax.experimental.pallas import tpu as pltpu

N_DEV = 8
M = 4096
N = 2048
NH = N // 2
CH = M // N_DEV
NSUB = 4
SUB = CH // NSUB


def kernel(x, w_mat):
    def body(x_ref, w_ref, out_ref, recv_a, recv_b, send_a, send_b,
             acc_a, acc_b, ag_a, ag_b, amax_buf, stage,
             rs_ssem_a, rs_rsem_a, rs_ssem_b, rs_rsem_b,
             ag_ssem_a, ag_rsem_a, ag_ssem_b, ag_rsem_b,
             am_ssem, am_rsem, copy_sem):
        i = lax.axis_index("i")

        def ring2dev(q):
            q = lax.rem(q + 2 * N_DEV, N_DEV)
            return jnp.where(q < 4, q, 11 - q)

        r = jnp.where(i < 4, i, 11 - i)
        left = ring2dev(r - 1)
        right = ring2dev(r + 1)

        barrier = pltpu.get_barrier_semaphore()
        for nbr in (left, right):
            pl.semaphore_signal(barrier, inc=1, device_id=(nbr,),
                                device_id_type=pl.DeviceIdType.MESH)
        pl.semaphore_wait(barrier, 2)

        def mk_rs(d, s, sub):
            rows = slice(sub * SUB, (sub + 1) * SUB)
            if d == 0:
                return pltpu.make_async_remote_copy(
                    src_ref=send_a.at[rows], dst_ref=recv_a.at[s, rows],
                    send_sem=rs_ssem_a.at[s, sub],
                    recv_sem=rs_rsem_a.at[s, sub],
                    device_id=(right,), device_id_type=pl.DeviceIdType.MESH)
            return pltpu.make_async_remote_copy(
                src_ref=send_b.at[rows], dst_ref=recv_b.at[s, rows],
                send_sem=rs_ssem_b.at[s, sub],
                recv_sem=rs_rsem_b.at[s, sub],
                device_id=(left,), device_id_type=pl.DeviceIdType.MESH)

        p_own = jnp.dot(x_ref[pl.ds(i * CH, CH), :], w_ref[...],
                        preferred_element_type=jnp.float32)
        send_a[...] = p_own[:, 0:NH].astype(jnp.bfloat16)
        send_b[...] = p_own[:, NH:N].astype(jnp.bfloat16)
        cur = {}
        for sub in range(NSUB):
            for d in (0, 1):
                desc = mk_rs(d, 0, sub)
                desc.start()
                cur[(d, sub)] = desc
        for s in range(N_DEV - 1):
            ca = ring2dev(r - s - 1)
            cb = ring2dev(r + s + 1)
            for sub in range(NSUB):
                rsl = slice(sub * SUB, (sub + 1) * SUB)
                da, db = cur[(0, sub)], cur[(1, sub)]
                pa = jnp.dot(x_ref[pl.ds(ca * CH + sub * SUB, SUB), :],
                             w_ref[:, 0:NH],
                             preferred_element_type=jnp.float32)
                pb = jnp.dot(x_ref[pl.ds(cb * CH + sub * SUB, SUB), :],
                             w_ref[:, NH:N],
                             preferred_element_type=jnp.float32)
                da.wait_recv()
                db.wait_recv()
                sum_a = pa + recv_a[s, rsl].astype(jnp.float32)
                sum_b = pb + recv_b[s, rsl].astype(jnp.float32)
                if s < N_DEV - 2:
                    da.wait_send()
                    db.wait_send()
                    send_a[rsl, :] = sum_a.astype(jnp.bfloat16)
                    na = mk_rs(0, s + 1, sub)
                    na.start()
                    cur[(0, sub)] = na
                    send_b[rsl, :] = sum_b.astype(jnp.bfloat16)
                    nb = mk_rs(1, s + 1, sub)
                    nb.start()
                    cur[(1, sub)] = nb
                else:
                    acc_a[rsl, :] = jnp.maximum(sum_a, 0.0)
                    acc_b[rsl, :] = jnp.maximum(sum_b, 0.0)
                    da.wait_send()
                    db.wait_send()

        am_local = jnp.maximum(jnp.max(acc_a[...]), jnp.max(acc_b[...]))
        amax_buf[pl.ds(i, 1), :] = jnp.full((1, 128), am_local, jnp.float32)
        send_descs = []
        for k in range(N_DEV):
            d = pltpu.make_async_remote_copy(
                src_ref=amax_buf.at[pl.ds(i, 1)],
                dst_ref=amax_buf.at[pl.ds(i, 1)],
                send_sem=am_ssem.at[k], recv_sem=am_rsem.at[i],
                device_id=(k,), device_id_type=pl.DeviceIdType.MESH)
            send_descs.append(d)

            @pl.when(i != k)
            def _(d=d):
                d.start()
        for k in range(N_DEV):
            rcv = pltpu.make_async_remote_copy(
                src_ref=amax_buf.at[pl.ds(k, 1)],
                dst_ref=amax_buf.at[pl.ds(k, 1)],
                send_sem=am_ssem.at[k], recv_sem=am_rsem.at[k],
                device_id=(k,), device_id_type=pl.DeviceIdType.MESH)

            @pl.when(i != k)
            def _(rcv=rcv, d=send_descs[k]):
                rcv.wait_recv()
                d.wait_send()
        amax = jnp.max(amax_buf[...])
        scale = amax / 448.0
        inv_scale = 448.0 / amax

        oa = right
        ob = left
        ag_a[pl.ds(oa * CH, CH), :] = (
            (acc_a[...] * inv_scale).astype(jnp.float8_e4m3fn))
        ag_b[pl.ds(ob * CH, CH), :] = (
            (acc_b[...] * inv_scale).astype(jnp.float8_e4m3fn))

        copies_a, copies_b = [], []

        def emit(rows, which):
            copies, base = (copies_a, 0) if which == 0 else (copies_b, 2)
            src = ag_a if which == 0 else ag_b
            col0 = 0 if which == 0 else NH
            j = len(copies)
            slot = base + (j % 2)
            if j >= 2:
                copies[j - 2].wait()
            stage[slot, :, :] = (
                src[rows, :].astype(jnp.float32) * scale
            ).astype(jnp.bfloat16)
            cp = pltpu.make_async_copy(
                stage.at[slot], out_ref.at[rows, col0:col0 + NH],
                copy_sem.at[slot])
            cp.start()
            copies.append(cp)

        emit(pl.ds(oa * CH, CH), 0)
        emit(pl.ds(ob * CH, CH), 1)

        def mk_ag(d, s, sub, chunk):
            rows = pl.ds(chunk * CH + sub * SUB, SUB)
            if d == 0:
                return pltpu.make_async_remote_copy(
                    src_ref=ag_a.at[rows], dst_ref=ag_a.at[rows],
                    send_sem=ag_ssem_a.at[s, sub],
                    recv_sem=ag_rsem_a.at[s, sub],
                    device_id=(right,), device_id_type=pl.DeviceIdType.MESH)
            return pltpu.make_async_remote_copy(
                src_ref=ag_b.at[rows], dst_ref=ag_b.at[rows],
                send_sem=ag_ssem_b.at[s, sub],
                recv_sem=ag_rsem_b.at[s, sub],
                device_id=(left,), device_id_type=pl.DeviceIdType.MESH)

        curg = {}
        for sub in range(NSUB):
            for d, own in ((0, oa), (1, ob)):
                desc = mk_ag(d, 0, sub, own)
                desc.start()
                curg[(d, sub)] = desc
        for s in range(N_DEV - 1):
            ra = ring2dev(r - s)
            rb = ring2dev(r + s)
            prev = []
            for sub in range(NSUB):
                da, db = curg[(0, sub)], curg[(1, sub)]
                da.wait_recv()
                db.wait_recv()
                prev += [da, db]
                if s < N_DEV - 2:
                    na = mk_ag(0, s + 1, sub, ra)
                    na.start()
                    curg[(0, sub)] = na
                    nb = mk_ag(1, s + 1, sub, rb)
                    nb.start()
                    curg[(1, sub)] = nb
            emit(pl.ds(ra * CH, CH), 0)
            emit(pl.ds(rb * CH, CH), 1)
            for dsc in prev:
                dsc.wait_send()

        for cp in copies_a[-2:]:
            cp.wait()
        for cp in copies_b[-2:]:
            cp.wait()

    nsem = N_DEV - 1
    return pl.pallas_call(
        body,
        out_shape=jax.ShapeDtypeStruct((M, N), jnp.bfloat16),
        in_specs=[pl.BlockSpec(memory_space=pltpu.VMEM),
                  pl.BlockSpec(memory_space=pltpu.VMEM)],
        out_specs=pl.BlockSpec(memory_space=pl.ANY),
        scratch_shapes=[
            pltpu.VMEM((nsem, CH, NH), jnp.bfloat16),
            pltpu.VMEM((nsem, CH, NH), jnp.bfloat16),
            pltpu.VMEM((CH, NH), jnp.bfloat16),
            pltpu.VMEM((CH, NH), jnp.bfloat16),
            pltpu.VMEM((CH, NH), jnp.float32),
            pltpu.VMEM((CH, NH), jnp.float32),
            pltpu.VMEM((M, NH), jnp.float8_e4m3fn),
            pltpu.VMEM((M, NH), jnp.float8_e4m3fn),
            pltpu.VMEM((N_DEV, 128), jnp.float32),
            pltpu.VMEM((4, CH, NH), jnp.bfloat16),
            pltpu.SemaphoreType.DMA((nsem, NSUB)),
            pltpu.SemaphoreType.DMA((nsem, NSUB)),
            pltpu.SemaphoreType.DMA((nsem, NSUB)),
            pltpu.SemaphoreType.DMA((nsem, NSUB)),
            pltpu.SemaphoreType.DMA((nsem, NSUB)),
            pltpu.SemaphoreType.DMA((nsem, NSUB)),
            pltpu.SemaphoreType.DMA((nsem, NSUB)),
            pltpu.SemaphoreType.DMA((nsem, NSUB)),
            pltpu.SemaphoreType.DMA((N_DEV,)),
            pltpu.SemaphoreType.DMA((N_DEV,)),
            pltpu.SemaphoreType.DMA((4,)),
        ],
        compiler_params=pltpu.CompilerParams(
            collective_id=0,
            vmem_limit_bytes=64 * 1024 * 1024,
        ),
    )(x, w_mat)


# device time: 152892 ns/iter; 1.0053x vs baseline; 1.0053x over previous
import jax
import jax.numpy as jnp
from jax import lax
from jax.experimental import pallas as pl
from jax.experimental.pallas import tpu as pltpu

N_DEV = 8
M = 4096
N = 2048
NH = N // 2
CH = M // N_DEV
SUB = CH // 2


def kernel(x, w_mat):
    def body(x_ref, w_ref, out_ref, recv_a, recv_b, send_a, send_b,
             acc_a, acc_b, ag_a, ag_b, amax_buf, stage,
             rs_ssem_a, rs_rsem_a, rs_ssem_b, rs_rsem_b,
             ag_ssem_a, ag_rsem_a, ag_ssem_b, ag_rsem_b,
             am_ssem, am_rsem, copy_sem):
        i = lax.axis_index("i")

        def ring2dev(q):
            q = lax.rem(q + 2 * N_DEV, N_DEV)
            return jnp.where(q < 4, q, 11 - q)

        r = jnp.where(i < 4, i, 11 - i)
        left = ring2dev(r - 1)
        right = ring2dev(r + 1)

        barrier = pltpu.get_barrier_semaphore()
        for nbr in (left, right):
            pl.semaphore_signal(barrier, inc=1, device_id=(nbr,),
                                device_id_type=pl.DeviceIdType.MESH)
        pl.semaphore_wait(barrier, 2)

        def mk_rs(d, s, sub):
            rows = slice(sub * SUB, (sub + 1) * SUB)
            if d == 0:
                return pltpu.make_async_remote_copy(
                    src_ref=send_a.at[rows], dst_ref=recv_a.at[s, rows],
                    send_sem=rs_ssem_a.at[s, sub],
                    recv_sem=rs_rsem_a.at[s, sub],
                    device_id=(right,), device_id_type=pl.DeviceIdType.MESH)
            return pltpu.make_async_remote_copy(
                src_ref=send_b.at[rows], dst_ref=recv_b.at[s, rows],
                send_sem=rs_ssem_b.at[s, sub],
                recv_sem=rs_rsem_b.at[s, sub],
                device_id=(left,), device_id_type=pl.DeviceIdType.MESH)

        p_own = jnp.dot(x_ref[pl.ds(i * CH, CH), :], w_ref[...],
                        preferred_element_type=jnp.float32)
        send_a[...] = p_own[:, 0:NH].astype(jnp.bfloat16)
        send_b[...] = p_own[:, NH:N].astype(jnp.bfloat16)
        cur = {}
        for sub in (0, 1):
            for d in (0, 1):
                desc = mk_rs(d, 0, sub)
                desc.start()
                cur[(d, sub)] = desc
        for s in range(N_DEV - 1):
            ca = ring2dev(r - s - 1)
            cb = ring2dev(r + s + 1)
            for sub in (0, 1):
                rsl = slice(sub * SUB, (sub + 1) * SUB)
                da, db = cur[(0, sub)], cur[(1, sub)]
                pa = jnp.dot(x_ref[pl.ds(ca * CH + sub * SUB, SUB), :],
                             w_ref[:, 0:NH],
                             preferred_element_type=jnp.float32)
                pb = jnp.dot(x_ref[pl.ds(cb * CH + sub * SUB, SUB), :],
                             w_ref[:, NH:N],
                             preferred_element_type=jnp.float32)
                da.wait_recv()
                db.wait_recv()
                sum_a = pa + recv_a[s, rsl].astype(jnp.float32)
                sum_b = pb + recv_b[s, rsl].astype(jnp.float32)
                if s < N_DEV - 2:
                    da.wait_send()
                    db.wait_send()
                    send_a[rsl, :] = sum_a.astype(jnp.bfloat16)
                    na = mk_rs(0, s + 1, sub)
                    na.start()
                    cur[(0, sub)] = na
                    send_b[rsl, :] = sum_b.astype(jnp.bfloat16)
                    nb = mk_rs(1, s + 1, sub)
                    nb.start()
                    cur[(1, sub)] = nb
                else:
                    acc_a[rsl, :] = jnp.maximum(sum_a, 0.0)
                    acc_b[rsl, :] = jnp.maximum(sum_b, 0.0)
                    da.wait_send()
                    db.wait_send()

        am_local = jnp.maximum(jnp.max(acc_a[...]), jnp.max(acc_b[...]))
        amax_buf[pl.ds(i, 1), :] = jnp.full((1, 128), am_local, jnp.float32)
        send_descs = []
        for k in range(N_DEV):
            d = pltpu.make_async_remote_copy(
                src_ref=amax_buf.at[pl.ds(i, 1)],
                dst_ref=amax_buf.at[pl.ds(i, 1)],
                send_sem=am_ssem.at[k], recv_sem=am_rsem.at[i],
                device_id=(k,), device_id_type=pl.DeviceIdType.MESH)
            send_descs.append(d)

            @pl.when(i != k)
            def _(d=d):
                d.start()
        for k in range(N_DEV):
            rcv = pltpu.make_async_remote_copy(
                src_ref=amax_buf.at[pl.ds(k, 1)],
                dst_ref=amax_buf.at[pl.ds(k, 1)],
                send_sem=am_ssem.at[k], recv_sem=am_rsem.at[k],
                device_id=(k,), device_id_type=pl.DeviceIdType.MESH)

            @pl.when(i != k)
            def _(rcv=rcv, d=send_descs[k]):
                rcv.wait_recv()
                d.wait_send()
        amax = jnp.max(amax_buf[...])
        scale = amax / 448.0
        inv_scale = 448.0 / amax

        oa = right
        ob = left
        ag_a[pl.ds(oa * CH, CH), :] = (
            (acc_a[...] * inv_scale).astype(jnp.float8_e4m3fn))
        ag_b[pl.ds(ob * CH, CH), :] = (
            (acc_b[...] * inv_scale).astype(jnp.float8_e4m3fn))

        copies_a, copies_b = [], []

        def emit(rows, which):
            copies, base = (copies_a, 0) if which == 0 else (copies_b, 2)
            src = ag_a if which == 0 else ag_b
            col0 = 0 if which == 0 else NH
            j = len(copies)
            slot = base + (j % 2)
            if j >= 2:
                copies[j - 2].wait()
            stage[slot, :, :] = (
                src[rows, :].astype(jnp.float32) * scale
            ).astype(jnp.bfloat16)
            cp = pltpu.make_async_copy(
                stage.at[slot], out_ref.at[rows, col0:col0 + NH],
                copy_sem.at[slot])
            cp.start()
            copies.append(cp)

        emit(pl.ds(oa * CH, CH), 0)
        emit(pl.ds(ob * CH, CH), 1)

        def mk_ag(d, s, sub, chunk):
            rows = pl.ds(chunk * CH + sub * SUB, SUB)
            if d == 0:
                return pltpu.make_async_remote_copy(
                    src_ref=ag_a.at[rows], dst_ref=ag_a.at[rows],
                    send_sem=ag_ssem_a.at[s, sub],
                    recv_sem=ag_rsem_a.at[s, sub],
                    device_id=(right,), device_id_type=pl.DeviceIdType.MESH)
            return pltpu.make_async_remote_copy(
                src_ref=ag_b.at[rows], dst_ref=ag_b.at[rows],
                send_sem=ag_ssem_b.at[s, sub],
                recv_sem=ag_rsem_b.at[s, sub],
                device_id=(left,), device_id_type=pl.DeviceIdType.MESH)

        curg = {}
        for sub in (0, 1):
            for d, own in ((0, oa), (1, ob)):
                desc = mk_ag(d, 0, sub, own)
                desc.start()
                curg[(d, sub)] = desc
        for s in range(N_DEV - 1):
            ra = ring2dev(r - s)
            rb = ring2dev(r + s)
            prev = []
            for sub in (0, 1):
                da, db = curg[(0, sub)], curg[(1, sub)]
                da.wait_recv()
                db.wait_recv()
                prev += [da, db]
                if s < N_DEV - 2:
                    na = mk_ag(0, s + 1, sub, ra)
                    na.start()
                    curg[(0, sub)] = na
                    nb = mk_ag(1, s + 1, sub, rb)
                    nb.start()
                    curg[(1, sub)] = nb
            emit(pl.ds(ra * CH, CH), 0)
            emit(pl.ds(rb * CH, CH), 1)
            for dsc in prev:
                dsc.wait_send()

        for cp in copies_a[-2:]:
            cp.wait()
        for cp in copies_b[-2:]:
            cp.wait()

    nsem = N_DEV - 1
    return pl.pallas_call(
        body,
        out_shape=jax.ShapeDtypeStruct((M, N), jnp.bfloat16),
        in_specs=[pl.BlockSpec(memory_space=pltpu.VMEM),
                  pl.BlockSpec(memory_space=pltpu.VMEM)],
        out_specs=pl.BlockSpec(memory_space=pl.ANY),
        scratch_shapes=[
            pltpu.VMEM((nsem, CH, NH), jnp.bfloat16),
            pltpu.VMEM((nsem, CH, NH), jnp.bfloat16),
            pltpu.VMEM((CH, NH), jnp.bfloat16),
            pltpu.VMEM((CH, NH), jnp.bfloat16),
            pltpu.VMEM((CH, NH), jnp.float32),
            pltpu.VMEM((CH, NH), jnp.float32),
            pltpu.VMEM((M, NH), jnp.float8_e4m3fn),
            pltpu.VMEM((M, NH), jnp.float8_e4m3fn),
            pltpu.VMEM((N_DEV, 128), jnp.float32),
            pltpu.VMEM((4, CH, NH), jnp.bfloat16),
            pltpu.SemaphoreType.DMA((nsem, 2)),
            pltpu.SemaphoreType.DMA((nsem, 2)),
            pltpu.SemaphoreType.DMA((nsem, 2)),
            pltpu.SemaphoreType.DMA((nsem, 2)),
            pltpu.SemaphoreType.DMA((nsem, 2)),
            pltpu.SemaphoreType.DMA((nsem, 2)),
            pltpu.SemaphoreType.DMA((nsem, 2)),
            pltpu.SemaphoreType.DMA((nsem, 2)),
            pltpu.SemaphoreType.DMA((N_DEV,)),
            pltpu.SemaphoreType.DMA((N_DEV,)),
            pltpu.SemaphoreType.DMA((4,)),
        ],
        compiler_params=pltpu.CompilerParams(
            collective_id=0,
            vmem_limit_bytes=64 * 1024 * 1024,
        ),
    )(x, w_mat)
